# baseline (speedup 1.0000x reference)
import jax
import jax.numpy as jnp
from jax.experimental import pallas as pl

G_SA00 = 0.003
G_SA01 = 0.001
G_SA10 = 0.002
G_SA11 = 3e-06

_R = 262144
_C = 128
_BR = 16384


def _phys_view(a):
    return (a.reshape(1024, 4, 128, 8, 8)
             .transpose(0, 3, 1, 4, 2)
             .reshape(_R, _C))


def _phys_unview(a):
    return (a.reshape(1024, 8, 4, 8, 128)
             .transpose(0, 2, 4, 1, 3)
             .reshape(1024, 512, 8, 8))


def _saf_body(x_ref, p_ref, o_ref):
    x = x_ref[...]
    p = p_ref[...]
    c = jnp.where(p == 1, G_SA00,
        jnp.where(p == 2, G_SA01,
        jnp.where(p == 3, G_SA10, G_SA11)))
    o_ref[...] = jnp.where(p == 0, x, c)


def kernel(input, mask, p_state):
    x = _phys_view(input)
    p = _phys_view(p_state)
    out = pl.pallas_call(
        _saf_body,
        out_shape=jax.ShapeDtypeStruct((_R, _C), jnp.float32),
        grid=(_R // _BR,),
        in_specs=[
            pl.BlockSpec((_BR, _C), lambda i: (i, 0)),
            pl.BlockSpec((_BR, _C), lambda i: (i, 0)),
        ],
        out_specs=pl.BlockSpec((_BR, _C), lambda i: (i, 0)),
    )(x, p)
    return _phys_unview(out)
